# baseline (speedup 1.0000x reference)
import jax
import jax.numpy as jnp
from jax import lax
from jax.experimental import pallas as pl
from jax.experimental.pallas import tpu as pltpu
from jax.experimental.pallas import tpu_sc as plsc

_ED = 64
_PD = 16
_NW = 32
_CHUNK = 128
_NBUF = 4


def _make_sc_gather(n_total, bsz):
    per_w = n_total // _NW
    nch = per_w // _CHUNK
    ch_per_h = bsz // _CHUNK
    mesh = plsc.VectorSubcoreMesh(core_axis_name="c", subcore_axis_name="s")

    def body(x_hbm, real_hbm, imag_hbm, real_out, imag_out, idx_v, *scr):
        bufr = scr[0:_NBUF]
        bufi = scr[_NBUF:2 * _NBUF]
        tbr = scr[2 * _NBUF:3 * _NBUF]
        tbi = scr[3 * _NBUF:4 * _NBUF]
        gsem = scr[4 * _NBUF:5 * _NBUF]
        osem = scr[5 * _NBUF:6 * _NBUF]

        info = plsc.get_sparse_core_info()
        wid = lax.axis_index("s") * info.num_cores + lax.axis_index("c")
        pltpu.sync_copy(x_hbm.at[pl.ds(wid * nch, nch), :], idx_v)

        iota = lax.iota(jnp.int32, 16)

        def fire_gather(j, b):
            pltpu.async_copy(real_hbm.at[idx_v.at[j]], bufr[b], gsem[b])
            pltpu.async_copy(imag_hbm.at[idx_v.at[j]], bufi[b], gsem[b])

        def drain_gather(b):
            pltpu.make_async_copy(real_hbm.at[pl.ds(0, _CHUNK)], bufr[b],
                                  gsem[b]).wait()
            pltpu.make_async_copy(imag_hbm.at[pl.ds(0, _CHUNK)], bufi[b],
                                  gsem[b]).wait()

        def transpose(b):
            def t_real(e, carry):
                for blk in range(_CHUNK // 16):
                    v = plsc.load_gather(bufr[b], [blk * 16 + iota, iota * 0 + e])
                    tbr[b][e, pl.ds(blk * 16, 16)] = v
                return carry
            lax.fori_loop(0, _ED, t_real, 0)

            def t_imag(e, carry):
                for blk in range(_CHUNK // 16):
                    v = plsc.load_gather(bufi[b], [blk * 16 + iota, iota * 0 + e])
                    tbi[b][e, pl.ds(blk * 16, 16)] = v
                return carry
            lax.fori_loop(0, _PD, t_imag, 0)

        def fire_out(j, b):
            c = wid * nch + j
            h = c // ch_per_h
            b0 = (c % ch_per_h) * _CHUNK
            pltpu.async_copy(tbr[b], real_out.at[pl.ds(h * _ED, _ED),
                                                 pl.ds(b0, _CHUNK)], osem[b])
            pltpu.async_copy(tbi[b], imag_out.at[pl.ds(h * _PD, _PD),
                                                 pl.ds(b0, _CHUNK)], osem[b])

        def drain_out(b):
            pltpu.make_async_copy(tbr[b], real_out.at[pl.ds(0, _ED), pl.ds(0, _CHUNK)],
                                  osem[b]).wait()
            pltpu.make_async_copy(tbi[b], imag_out.at[pl.ds(0, _PD), pl.ds(0, _CHUNK)],
                                  osem[b]).wait()

        for b in range(_NBUF):
            fire_gather(b, b)

        def outer(j0, carry):
            for b in range(_NBUF):
                j = j0 * _NBUF + b
                drain_gather(b)

                @pl.when(j >= _NBUF)
                def _():
                    drain_out(b)

                transpose(b)

                @pl.when(j + _NBUF < nch)
                def _():
                    fire_gather(j + _NBUF, b)

                fire_out(j, b)
            return carry

        lax.fori_loop(0, nch // _NBUF, outer, 0)
        for b in range(_NBUF):
            drain_out(b)

    return pl.kernel(
        body,
        out_type=(
            jax.ShapeDtypeStruct((n_total // bsz * _ED, bsz), jnp.float32),
            jax.ShapeDtypeStruct((n_total // bsz * _PD, bsz), jnp.float32),
        ),
        mesh=mesh,
        scratch_types=(
            [pltpu.VMEM((nch, _CHUNK), jnp.int32)]
            + [pltpu.VMEM((_CHUNK, _ED), jnp.float32)] * _NBUF
            + [pltpu.VMEM((_CHUNK, _PD), jnp.float32)] * _NBUF
            + [pltpu.VMEM((_ED, _CHUNK), jnp.float32)] * _NBUF
            + [pltpu.VMEM((_PD, _CHUNK), jnp.float32)] * _NBUF
            + [pltpu.SemaphoreType.DMA] * (2 * _NBUF)
        ),
        compiler_params=pltpu.CompilerParams(use_tc_tiling_on_sc=False,
                                             needs_layout_passes=False),
    )


def kernel(x, real_table, imag_table):
    bsz, hist = x.shape
    n = bsz * hist
    vocab = real_table.shape[0]
    xt = jnp.transpose(x).reshape(n // _CHUNK, _CHUNK).astype(jnp.int32)
    rt = lax.optimization_barrier(real_table.reshape(-1)).reshape(vocab, _ED)
    it = lax.optimization_barrier(imag_table.reshape(-1)).reshape(vocab, _PD)
    r2, i2 = _make_sc_gather(n, bsz)(xt, rt, it)
    real = r2.reshape(hist, _ED, bsz).transpose(2, 0, 1)
    imag = i2.reshape(hist, _PD, bsz).transpose(2, 0, 1)
    return (real, imag)


# speedup vs baseline: 1.6030x; 1.6030x over previous
import jax
import jax.numpy as jnp
from jax import lax
from jax.experimental import pallas as pl
from jax.experimental.pallas import tpu as pltpu
from jax.experimental.pallas import tpu_sc as plsc

_ED = 64
_PD = 16
_NW = 32
_CHUNK = 128
_NBUF = 2


def _make_sc_gather(n_total, bsz):
    per_w = n_total // _NW
    nch = per_w // _CHUNK
    ch_per_h = bsz // _CHUNK
    mesh = plsc.VectorSubcoreMesh(core_axis_name="c", subcore_axis_name="s")

    def body(x_hbm, real_hbm, imag_hbm, real_out, imag_out, *scr):
        bufr = scr[0:_NBUF]
        bufi = scr[_NBUF:2 * _NBUF]
        tbr = scr[2 * _NBUF:3 * _NBUF]
        tbi = scr[3 * _NBUF:4 * _NBUF]
        idxr = scr[4 * _NBUF]
        ihf = scr[4 * _NBUF + 1]
        iqt = scr[4 * _NBUF + 2]
        gsem = scr[4 * _NBUF + 3:5 * _NBUF + 3]
        osem = scr[5 * _NBUF + 3:6 * _NBUF + 3]

        info = plsc.get_sparse_core_info()
        wid = lax.axis_index("s") * info.num_cores + lax.axis_index("c")
        iota = lax.iota(jnp.int32, 16)

        def load_idx(j, b):
            pltpu.sync_copy(x_hbm.at[pl.ds(wid * nch + j, 1), :],
                            idxr.at[pl.ds(b, 1), :])
            for blk in range(_CHUNK // 16):
                v = idxr[b, pl.ds(blk * 16, 16)]
                ihf[b, pl.ds(blk * 16, 16)] = lax.shift_right_logical(v, 1)
                iqt[b, pl.ds(blk * 16, 16)] = lax.shift_right_logical(v, 3)

        def fire_gather(b):
            pltpu.async_copy(real_hbm.at[ihf.at[b]], bufr[b], gsem[b])
            pltpu.async_copy(imag_hbm.at[iqt.at[b]], bufi[b], gsem[b])

        def drain_gather(b):
            pltpu.make_async_copy(real_hbm.at[pl.ds(0, _CHUNK)], bufr[b],
                                  gsem[b]).wait()
            pltpu.make_async_copy(imag_hbm.at[pl.ds(0, _CHUNK)], bufi[b],
                                  gsem[b]).wait()

        def transpose(b):
            colr = []
            coli = []
            for ib in range(_CHUNK // 16):
                v = idxr[b, pl.ds(ib * 16, 16)]
                colr.append((v & 1) * _ED)
                coli.append((v & 7) * _PD)

            def diag(d, carry):
                rot = (iota + d) & 15
                for ib in range(_CHUNK // 16):
                    ivec = ib * 16 + iota
                    for eb in range(_ED // 16):
                        c = colr[ib] + eb * 16 + rot
                        g = plsc.load_gather(bufr[b], [ivec, c])
                        plsc.store_scatter(tbr[b], [eb * 16 + rot, ivec], g)
                    g = plsc.load_gather(bufi[b], [ivec, coli[ib] + rot])
                    plsc.store_scatter(tbi[b], [rot, ivec], g)
                return carry
            lax.fori_loop(0, 16, diag, 0)

        def fire_out(j, b):
            c = wid * nch + j
            h = c // ch_per_h
            bt = c % ch_per_h
            for et in range(_ED // 8):
                row = ((h * (_ED // 8) + et) * ch_per_h + bt) * 8
                pltpu.async_copy(tbr[b].at[pl.ds(et * 8, 8), :],
                                 real_out.at[pl.ds(row, 8), :], osem[b])
            for et in range(_PD // 8):
                row = ((h * (_PD // 8) + et) * ch_per_h + bt) * 8
                pltpu.async_copy(tbi[b].at[pl.ds(et * 8, 8), :],
                                 imag_out.at[pl.ds(row, 8), :], osem[b])

        def drain_out(b):
            pltpu.make_async_copy(tbr[b], real_out.at[pl.ds(0, _ED), :],
                                  osem[b]).wait()
            pltpu.make_async_copy(tbi[b], imag_out.at[pl.ds(0, _PD), :],
                                  osem[b]).wait()

        for b in range(_NBUF):
            load_idx(b, b)
            fire_gather(b)

        def outer(j0, carry):
            for b in range(_NBUF):
                j = j0 * _NBUF + b
                drain_gather(b)

                @pl.when(j >= _NBUF)
                def _():
                    drain_out(b)

                transpose(b)
                fire_out(j, b)

                @pl.when(j + _NBUF < nch)
                def _():
                    load_idx(j + _NBUF, b)
                    fire_gather(b)
            return carry

        lax.fori_loop(0, nch // _NBUF, outer, 0)
        for b in range(_NBUF):
            drain_out(b)

    return pl.kernel(
        body,
        out_type=(
            jax.ShapeDtypeStruct((n_total * _ED // 128, 128), jnp.float32),
            jax.ShapeDtypeStruct((n_total * _PD // 128, 128), jnp.float32),
        ),
        mesh=mesh,
        scratch_types=(
            [pltpu.VMEM((_CHUNK, _CHUNK), jnp.float32)] * _NBUF
            + [pltpu.VMEM((_CHUNK, _CHUNK), jnp.float32)] * _NBUF
            + [pltpu.VMEM((_ED, _CHUNK), jnp.float32)] * _NBUF
            + [pltpu.VMEM((_PD, _CHUNK), jnp.float32)] * _NBUF
            + [pltpu.VMEM((_NBUF, _CHUNK), jnp.int32)] * 3
            + [pltpu.SemaphoreType.DMA] * (2 * _NBUF)
        ),
        compiler_params=pltpu.CompilerParams(use_tc_tiling_on_sc=False,
                                             needs_layout_passes=False),
    )


def kernel(x, real_table, imag_table):
    bsz, hist = x.shape
    n = bsz * hist
    vocab = real_table.shape[0]
    xt = jnp.transpose(x).reshape(n // _CHUNK, _CHUNK).astype(jnp.int32)
    rt = real_table.reshape(vocab // 2, 128)
    it = imag_table.reshape(vocab // 8, 128)
    r1, i1 = _make_sc_gather(n, bsz)(xt, rt, it)
    real = (r1.reshape(hist, _ED // 8, bsz // _CHUNK, 8, _CHUNK)
            .transpose(2, 4, 0, 1, 3).reshape(bsz, hist, _ED))
    imag = (i1.reshape(hist, _PD // 8, bsz // _CHUNK, 8, _CHUNK)
            .transpose(2, 4, 0, 1, 3).reshape(bsz, hist, _PD))
    return (real, imag)


# speedup vs baseline: 1.6733x; 1.0439x over previous
import jax
import jax.numpy as jnp
from jax import lax
from jax.experimental import pallas as pl
from jax.experimental.pallas import tpu as pltpu
from jax.experimental.pallas import tpu_sc as plsc

_ED = 64
_PD = 16
_NW = 32
_CHUNK = 128
_NBUF = 4


def _make_sc_gather(n_total, bsz):
    per_w = n_total // _NW
    nch = per_w // _CHUNK
    ch_per_h = bsz // _CHUNK
    mesh = plsc.VectorSubcoreMesh(core_axis_name="c", subcore_axis_name="s")

    def body(x_hbm, real_hbm, imag_hbm, real_out, imag_out, *scr):
        bufr = scr[0:_NBUF]
        bufi = scr[_NBUF:2 * _NBUF]
        tbr = scr[2 * _NBUF:3 * _NBUF]
        tbi = scr[3 * _NBUF:4 * _NBUF]
        idxr = scr[4 * _NBUF]
        gsem = scr[4 * _NBUF + 1:5 * _NBUF + 1]
        osem = scr[5 * _NBUF + 1:6 * _NBUF + 1]

        info = plsc.get_sparse_core_info()
        wid = lax.axis_index("s") * info.num_cores + lax.axis_index("c")
        iota = lax.iota(jnp.int32, 16)

        def load_idx(j, b):
            pltpu.sync_copy(x_hbm.at[pl.ds(wid * nch + j, 1), :],
                            idxr.at[pl.ds(b, 1), :])

        def fire_gather(b):
            pltpu.async_copy(real_hbm.at[idxr.at[b]], bufr[b], gsem[b])
            pltpu.async_copy(imag_hbm.at[idxr.at[b]], bufi[b], gsem[b])

        def drain_gather(b):
            pltpu.make_async_copy(real_hbm.at[pl.ds(0, _CHUNK)], bufr[b],
                                  gsem[b]).wait()
            pltpu.make_async_copy(imag_hbm.at[pl.ds(0, _CHUNK)], bufi[b],
                                  gsem[b]).wait()

        def transpose(b):
            def diag(d, carry):
                rot = (iota + d) & 15
                for ib in range(_CHUNK // 16):
                    ivec = ib * 16 + iota
                    for eb in range(_ED // 16):
                        g = plsc.load_gather(bufr[b], [ivec, eb * 16 + rot])
                        plsc.store_scatter(tbr[b], [eb * 16 + rot, ivec], g)
                    g = plsc.load_gather(bufi[b], [ivec, rot])
                    plsc.store_scatter(tbi[b], [rot, ivec], g)
                return carry
            lax.fori_loop(0, 16, diag, 0)

        def fire_out(j, b):
            c = wid * nch + j
            h = c // ch_per_h
            bt = c % ch_per_h
            for et in range(_ED // 8):
                row = ((h * (_ED // 8) + et) * ch_per_h + bt) * 8
                pltpu.async_copy(tbr[b].at[pl.ds(et * 8, 8), :],
                                 real_out.at[pl.ds(row, 8), :], osem[b])
            for et in range(_PD // 8):
                row = ((h * (_PD // 8) + et) * ch_per_h + bt) * 8
                pltpu.async_copy(tbi[b].at[pl.ds(et * 8, 8), :],
                                 imag_out.at[pl.ds(row, 8), :], osem[b])

        def drain_out(b):
            pltpu.make_async_copy(tbr[b], real_out.at[pl.ds(0, _ED), :],
                                  osem[b]).wait()
            pltpu.make_async_copy(tbi[b], imag_out.at[pl.ds(0, _PD), :],
                                  osem[b]).wait()

        for b in range(_NBUF):
            load_idx(b, b)
            fire_gather(b)

        def outer(j0, carry):
            for b in range(_NBUF):
                j = j0 * _NBUF + b
                drain_gather(b)

                @pl.when(j >= _NBUF)
                def _():
                    drain_out(b)

                transpose(b)
                fire_out(j, b)

                @pl.when(j + _NBUF < nch)
                def _():
                    load_idx(j + _NBUF, b)
                    fire_gather(b)
            return carry

        lax.fori_loop(0, nch // _NBUF, outer, 0)
        for b in range(_NBUF):
            drain_out(b)

    return pl.kernel(
        body,
        out_type=(
            jax.ShapeDtypeStruct((n_total * _ED // 128, 128), jnp.float32),
            jax.ShapeDtypeStruct((n_total * _PD // 128, 128), jnp.float32),
        ),
        mesh=mesh,
        scratch_types=(
            [pltpu.VMEM((_CHUNK, _ED), jnp.float32)] * _NBUF
            + [pltpu.VMEM((_CHUNK, _PD), jnp.float32)] * _NBUF
            + [pltpu.VMEM((_ED, _CHUNK), jnp.float32)] * _NBUF
            + [pltpu.VMEM((_PD, _CHUNK), jnp.float32)] * _NBUF
            + [pltpu.VMEM((_NBUF, _CHUNK), jnp.int32)]
            + [pltpu.SemaphoreType.DMA] * (2 * _NBUF)
        ),
        compiler_params=pltpu.CompilerParams(use_tc_tiling_on_sc=False,
                                             needs_layout_passes=False),
    )


def kernel(x, real_table, imag_table):
    bsz, hist = x.shape
    n = bsz * hist
    vocab = real_table.shape[0]
    xt = jnp.transpose(x).reshape(n // _CHUNK, _CHUNK).astype(jnp.int32)
    rt = real_table
    it = imag_table
    r1, i1 = _make_sc_gather(n, bsz)(xt, rt, it)
    real = (r1.reshape(hist, _ED // 8, bsz // _CHUNK, 8, _CHUNK)
            .transpose(2, 4, 0, 1, 3).reshape(bsz, hist, _ED))
    imag = (i1.reshape(hist, _PD // 8, bsz // _CHUNK, 8, _CHUNK)
            .transpose(2, 4, 0, 1, 3).reshape(bsz, hist, _PD))
    return (real, imag)
